# baseline (speedup 1.0000x reference)
import functools

import jax
import jax.numpy as jnp
from jax import lax
from jax.experimental import pallas as pl
from jax.experimental.pallas import tpu as pltpu
from jax.experimental.pallas import tpu_sc as plsc

N = 10000
E = 320000
NC, NS = 2, 16
NW = NC * NS
C = 128
NCH = 80
EPT = NCH * C
EPAD = NW * EPT
NPAD = 10240
RPT = NPAD // NS
NB = 8
GRPS = NCH // NB
_BM = 2048

_MESH = plsc.VectorSubcoreMesh(core_axis_name="c", subcore_axis_name="s")
_SC_PARAMS = pltpu.CompilerParams(use_tc_tiling_on_sc=False)


def _make_prop(FB, ncb):
    @functools.partial(
        pl.kernel,
        out_type=jax.ShapeDtypeStruct((NPAD, 128), jnp.float32),
        mesh=_MESH,
        compiler_params=_SC_PARAMS,
        scratch_types=[
            pltpu.VMEM((NCH, C), jnp.int32),
            pltpu.VMEM((NCH, C), jnp.int32),
            pltpu.VMEM((2, NB, C, FB), jnp.float32),
            pltpu.VMEM((C, FB), jnp.float32),
            pltpu.SemaphoreType.DMA,
            pltpu.SemaphoreType.DMA,
            pltpu.SemaphoreType.DMA,
            pltpu.SemaphoreType.DMA,
            pltpu.SemaphoreType.DMA,
            pltpu.VMEM_SHARED((NPAD, FB), jnp.float32),
            pltpu.VMEM_SHARED((NPAD, FB), jnp.float32),
        ],
    )
    def prop(h_hbm, src_hbm, dst_hbm, out_hbm, srcv, dstv, rows, zbuf,
             g0, g1, s0, s1, hsem, acc, hsp):
        cid = lax.axis_index("c")
        sid = lax.axis_index("s")
        wid = sid * NC + cid
        gsem = (g0, g1)
        ssem = (s0, s1)

        pltpu.sync_copy(src_hbm.at[wid], srcv)
        pltpu.sync_copy(dst_hbm.at[wid], dstv)

        def zrow(i, carry):
            for c4 in range(FB // 16):
                zbuf[i, pl.ds(c4 * 16, 16)] = jnp.zeros((16,), jnp.float32)
            return carry

        lax.fori_loop(0, C, zrow, 0)

        def fire_g(slot, grp):
            for b in range(NB):
                i = grp * NB + b
                pltpu.async_copy(hsp.at[srcv.at[i]], rows.at[slot, b],
                                 gsem[slot])

        def wait_g(slot, grp):
            for b in range(NB):
                i = grp * NB + b
                pltpu.make_async_copy(hsp.at[srcv.at[i]],
                                      rows.at[slot, b], gsem[slot]).wait()

        def run_s(slot, grp):
            ds = []
            for b in range(NB):
                i = grp * NB + b
                ds.append(pltpu.async_copy(rows.at[slot, b],
                                           acc.at[dstv.at[i]], ssem[slot],
                                           add=True))
            for d in ds:
                d.wait()

        for cb in range(ncb):
            stg = pltpu.async_copy(
                h_hbm.at[pl.ds(sid * RPT, RPT), pl.ds(cb * FB, FB)],
                hsp.at[pl.ds(sid * RPT, RPT)], hsem)
            for r in range(RPT // C):
                pltpu.sync_copy(zbuf, acc.at[pl.ds(sid * RPT + r * C, C)])
            stg.wait()
            plsc.subcore_barrier()

            fire_g(0, 0)
            fire_g(1, 1)

            def outer(j2, carry):
                j = j2 * 2
                wait_g(0, j)
                run_s(0, j)
                fire_g(0, lax.rem(j + 2, GRPS))
                wait_g(1, j + 1)
                run_s(1, j + 1)
                fire_g(1, lax.rem(j + 3, GRPS))
                return carry

            lax.fori_loop(0, GRPS // 2, outer, 0)
            wait_g(0, 0)
            wait_g(1, 1)
            plsc.subcore_barrier()
            pltpu.sync_copy(
                acc.at[pl.ds(sid * RPT, RPT)],
                out_hbm.at[pl.ds(sid * RPT, RPT),
                           pl.ds((cb * NC + cid) * FB, FB)],
            )
            if cb + 1 < ncb:
                plsc.subcore_barrier()

    return prop


_prop64 = _make_prop(32, 2)
_prop32 = _make_prop(32, 1)
_prop16 = _make_prop(16, 1)


@functools.partial(
    pl.kernel,
    out_type=jax.ShapeDtypeStruct((2 * NPAD,), jnp.float32),
    mesh=_MESH,
    compiler_params=_SC_PARAMS,
    scratch_types=[
        pltpu.VMEM((NCH, C), jnp.int32),
        pltpu.VMEM((C,), jnp.float32),
        pltpu.SemaphoreType.DMA,
        pltpu.VMEM_SHARED((NPAD,), jnp.float32),
    ],
)
def _deg_pass(dst_hbm, out_hbm, dstv, ones, sem, acc):
    cid = lax.axis_index("c")
    sid = lax.axis_index("s")
    wid = sid * NC + cid

    pltpu.sync_copy(dst_hbm.at[wid], dstv)
    for c4 in range(C // 16):
        ones[pl.ds(c4 * 16, 16)] = jnp.zeros((16,), jnp.float32)
    for r in range(RPT // C):
        pltpu.sync_copy(ones, acc.at[pl.ds(sid * RPT + r * C, C)])
    plsc.subcore_barrier()
    for c4 in range(C // 16):
        ones[pl.ds(c4 * 16, 16)] = jnp.ones((16,), jnp.float32)

    def body(j, carry):
        ds = []
        for b in range(8):
            i = j * 8 + b
            ds.append(pltpu.async_copy(ones, acc.at[dstv.at[i]], sem,
                                       add=True))
        for d in ds:
            d.wait()
        return carry

    lax.fori_loop(0, NCH // 8, body, 0)
    plsc.subcore_barrier()
    pltpu.sync_copy(
        acc.at[pl.ds(sid * RPT, RPT)],
        out_hbm.at[pl.ds(cid * NPAD + sid * RPT, RPT)],
    )


def _blk(w, bm=_BM):
    return pl.BlockSpec((bm, w), lambda i: (i, 0))


def _full(r, c):
    return pl.BlockSpec((r, c), lambda i: (0, 0))


def _mm1s(x, W, degp):
    M, K = x.shape
    F = W.shape[1]

    def body(xr, wr, d0r, d1r, o):
        dinv = lax.rsqrt(d0r[...] + d1r[...] + 1.0)
        h = dinv * jnp.dot(xr[...], wr[...],
                           preferred_element_type=jnp.float32)
        o[...] = jnp.concatenate(
            [h, dinv, jnp.zeros((h.shape[0], 128 - F - 1), jnp.float32)],
            axis=1)

    nsec = NPAD // _BM
    return pl.pallas_call(
        body,
        grid=(M // _BM,),
        in_specs=[
            _blk(K), _full(K, F),
            pl.BlockSpec((_BM, 1), lambda i: (i, 0)),
            pl.BlockSpec((_BM, 1), lambda i: (nsec + i, 0)),
        ],
        out_specs=_blk(128),
        out_shape=jax.ShapeDtypeStruct((M, 128), jnp.float32),
    )(x, W, degp, degp)


def _layer1(a, p1, b, W):
    F2 = W.shape[1]

    def body(ar, pr, br, wr, o):
        av = ar[...]
        pv = pr[...]
        dinv = pv[:, 64:65]
        acc = jnp.concatenate(
            [av[:, 0:32] + av[:, 32:64], av[:, 64:96] + av[:, 96:128]],
            axis=1)
        t = dinv * (acc + pv[:, 0:64]) + br[...]
        t = jnp.maximum(t, 0.0)
        h = dinv * jnp.dot(t, wr[...], preferred_element_type=jnp.float32)
        o[...] = jnp.concatenate(
            [h, jnp.zeros((h.shape[0], 128 - F2), jnp.float32)], axis=1)

    return pl.pallas_call(
        body,
        grid=(NPAD // _BM,),
        in_specs=[_blk(128), _blk(128), _full(1, 64), _full(64, F2)],
        out_specs=_blk(128),
        out_shape=jax.ShapeDtypeStruct((NPAD, 128), jnp.float32),
    )(a, p1, b, W)


def _layer2(a, p2, p1, b, W):
    F = 32
    F2 = W.shape[1]

    def body(ar, p2r, p1r, br, wr, o):
        av = ar[...]
        dinv = p1r[...][:, 64:65]
        t = dinv * (av[:, 0:F] + av[:, F:2 * F] + p2r[...][:, 0:F]) + br[...]
        t = jnp.maximum(t, 0.0)
        h = dinv * jnp.dot(t, wr[...], preferred_element_type=jnp.float32)
        o[...] = jnp.concatenate(
            [h, jnp.zeros((h.shape[0], 128 - F2), jnp.float32)], axis=1)

    return pl.pallas_call(
        body,
        grid=(NPAD // _BM,),
        in_specs=[
            _blk(128), _blk(128), _blk(128),
            _full(1, F), _full(F, F2),
        ],
        out_specs=_blk(128),
        out_shape=jax.ShapeDtypeStruct((NPAD, 128), jnp.float32),
    )(a, p2, p1, b, W)


def _final(a, p3, p1, b):
    F = 16
    bm = 1000

    def body(ar, p3r, p1r, br, o):
        av = ar[...]
        dinv = p1r[...][:, 64:65]
        t = dinv * (av[:, 0:F] + av[:, F:2 * F] + p3r[...][:, 0:F]) + br[...]
        o[...] = jax.nn.sigmoid(t)

    return pl.pallas_call(
        body,
        grid=(N // bm,),
        in_specs=[
            _blk(128, bm), _blk(128, bm), _blk(128, bm),
            _full(1, F),
        ],
        out_specs=_blk(F, bm),
        out_shape=jax.ShapeDtypeStruct((N, F), jnp.float32),
    )(a, p3, p1, b)


def kernel(x, edge_index, W1, b1, W2, b2, W3, b3):
    ei = edge_index.astype(jnp.int32)
    pad = EPAD - E
    src = jnp.concatenate([ei[0], jnp.zeros((pad,), jnp.int32)])
    dst = jnp.concatenate(
        [ei[1], N + (jnp.arange(pad, dtype=jnp.int32) % (NPAD - N))])
    src = src.reshape(NW, NCH, C)
    dst = dst.reshape(NW, NCH, C)
    xp = jnp.pad(x, ((0, NPAD - N), (0, 0)))

    degp = _deg_pass(dst).reshape(2 * NPAD, 1)
    p1 = _mm1s(xp, W1, degp)

    a = _prop64(p1, src, dst)
    p2 = _layer1(a, p1, b1.reshape(1, -1), W2)

    a = _prop32(p2, src, dst)
    p3 = _layer2(a, p2, p1, b2.reshape(1, -1), W3)

    a = _prop16(p3, src, dst)
    return _final(a, p3, p1, b3.reshape(1, -1))


# speedup vs baseline: 1.0155x; 1.0155x over previous
import functools

import jax
import jax.numpy as jnp
from jax import lax
from jax.experimental import pallas as pl
from jax.experimental.pallas import tpu as pltpu
from jax.experimental.pallas import tpu_sc as plsc

N = 10000
E = 320000
NC, NS = 2, 16
NW = NC * NS
C = 128
NCH = 80
EPT = NCH * C
EPAD = NW * EPT
NPAD = 10240
RPT = NPAD // NS
NB = 4
GRPS = NCH // NB
_BM = 2048

_MESH = plsc.VectorSubcoreMesh(core_axis_name="c", subcore_axis_name="s")
_SC_PARAMS = pltpu.CompilerParams(use_tc_tiling_on_sc=False)


def _make_prop(FB, ncb):
    @functools.partial(
        pl.kernel,
        out_type=jax.ShapeDtypeStruct((NPAD, 128), jnp.float32),
        mesh=_MESH,
        compiler_params=_SC_PARAMS,
        scratch_types=[
            pltpu.VMEM((NCH, C), jnp.int32),
            pltpu.VMEM((NCH, C), jnp.int32),
            pltpu.VMEM((2, NB, C, FB), jnp.float32),
            pltpu.VMEM((C, FB), jnp.float32),
            pltpu.SemaphoreType.DMA,
            pltpu.SemaphoreType.DMA,
            pltpu.SemaphoreType.DMA,
            pltpu.SemaphoreType.DMA,
            pltpu.SemaphoreType.DMA,
            pltpu.VMEM_SHARED((NPAD, FB), jnp.float32),
            pltpu.VMEM_SHARED((NPAD, FB), jnp.float32),
        ],
    )
    def prop(h_hbm, src_hbm, dst_hbm, out_hbm, srcv, dstv, rows, zbuf,
             g0, g1, s0, s1, hsem, acc, hsp):
        cid = lax.axis_index("c")
        sid = lax.axis_index("s")
        wid = sid * NC + cid
        gsem = (g0, g1)
        ssem = (s0, s1)

        pltpu.sync_copy(src_hbm.at[wid], srcv)
        pltpu.sync_copy(dst_hbm.at[wid], dstv)

        def zrow(i, carry):
            for c4 in range(FB // 16):
                zbuf[i, pl.ds(c4 * 16, 16)] = jnp.zeros((16,), jnp.float32)
            return carry

        lax.fori_loop(0, C, zrow, 0)

        def fire_g(slot, grp):
            for b in range(NB):
                i = grp * NB + b
                pltpu.async_copy(hsp.at[srcv.at[i]], rows.at[slot, b],
                                 gsem[slot])

        def wait_g(slot, grp):
            for b in range(NB):
                i = grp * NB + b
                pltpu.make_async_copy(hsp.at[srcv.at[i]],
                                      rows.at[slot, b], gsem[slot]).wait()

        def run_s(slot, grp):
            ds = []
            for b in range(NB):
                i = grp * NB + b
                ds.append(pltpu.async_copy(rows.at[slot, b],
                                           acc.at[dstv.at[i]], ssem[slot],
                                           add=True))
            for d in ds:
                d.wait()

        for cb in range(ncb):
            stg = pltpu.async_copy(
                h_hbm.at[pl.ds(sid * RPT, RPT), pl.ds(cb * FB, FB)],
                hsp.at[pl.ds(sid * RPT, RPT)], hsem)
            for r in range(RPT // C):
                pltpu.sync_copy(zbuf, acc.at[pl.ds(sid * RPT + r * C, C)])
            stg.wait()
            plsc.subcore_barrier()

            fire_g(0, 0)
            fire_g(1, 1)

            def outer(j2, carry):
                j = j2 * 2
                wait_g(0, j)
                run_s(0, j)
                fire_g(0, lax.rem(j + 2, GRPS))
                wait_g(1, j + 1)
                run_s(1, j + 1)
                fire_g(1, lax.rem(j + 3, GRPS))
                return carry

            lax.fori_loop(0, GRPS // 2, outer, 0)
            wait_g(0, 0)
            wait_g(1, 1)
            plsc.subcore_barrier()
            pltpu.sync_copy(
                acc.at[pl.ds(sid * RPT, RPT)],
                out_hbm.at[pl.ds(sid * RPT, RPT),
                           pl.ds((cb * NC + cid) * FB, FB)],
            )
            if cb + 1 < ncb:
                plsc.subcore_barrier()

    return prop


_prop64 = _make_prop(32, 2)
_prop32 = _make_prop(32, 1)
_prop16 = _make_prop(16, 1)


@functools.partial(
    pl.kernel,
    out_type=jax.ShapeDtypeStruct((2 * NPAD,), jnp.float32),
    mesh=_MESH,
    compiler_params=_SC_PARAMS,
    scratch_types=[
        pltpu.VMEM((NCH, C), jnp.int32),
        pltpu.VMEM((C,), jnp.float32),
        pltpu.SemaphoreType.DMA,
        pltpu.VMEM_SHARED((NPAD,), jnp.float32),
    ],
)
def _deg_pass(dst_hbm, out_hbm, dstv, ones, sem, acc):
    cid = lax.axis_index("c")
    sid = lax.axis_index("s")
    wid = sid * NC + cid

    pltpu.sync_copy(dst_hbm.at[wid], dstv)
    for c4 in range(C // 16):
        ones[pl.ds(c4 * 16, 16)] = jnp.zeros((16,), jnp.float32)
    for r in range(RPT // C):
        pltpu.sync_copy(ones, acc.at[pl.ds(sid * RPT + r * C, C)])
    plsc.subcore_barrier()
    for c4 in range(C // 16):
        ones[pl.ds(c4 * 16, 16)] = jnp.ones((16,), jnp.float32)

    def body(j, carry):
        ds = []
        for b in range(8):
            i = j * 8 + b
            ds.append(pltpu.async_copy(ones, acc.at[dstv.at[i]], sem,
                                       add=True))
        for d in ds:
            d.wait()
        return carry

    lax.fori_loop(0, NCH // 8, body, 0)
    plsc.subcore_barrier()
    pltpu.sync_copy(
        acc.at[pl.ds(sid * RPT, RPT)],
        out_hbm.at[pl.ds(cid * NPAD + sid * RPT, RPT)],
    )


def _blk(w, bm=_BM):
    return pl.BlockSpec((bm, w), lambda i: (i, 0))


def _full(r, c):
    return pl.BlockSpec((r, c), lambda i: (0, 0))


def _mm1s(x, W, degp):
    M, K = x.shape
    F = W.shape[1]

    def body(xr, wr, d0r, d1r, o):
        dinv = lax.rsqrt(d0r[...] + d1r[...] + 1.0)
        h = dinv * jnp.dot(xr[...], wr[...],
                           preferred_element_type=jnp.float32)
        o[...] = jnp.concatenate(
            [h, dinv, jnp.zeros((h.shape[0], 128 - F - 1), jnp.float32)],
            axis=1)

    nsec = NPAD // _BM
    return pl.pallas_call(
        body,
        grid=(M // _BM,),
        in_specs=[
            _blk(K), _full(K, F),
            pl.BlockSpec((_BM, 1), lambda i: (i, 0)),
            pl.BlockSpec((_BM, 1), lambda i: (nsec + i, 0)),
        ],
        out_specs=_blk(128),
        out_shape=jax.ShapeDtypeStruct((M, 128), jnp.float32),
    )(x, W, degp, degp)


def _layer1(a, p1, b, W):
    F2 = W.shape[1]

    def body(ar, pr, br, wr, o):
        av = ar[...]
        pv = pr[...]
        dinv = pv[:, 64:65]
        acc = jnp.concatenate(
            [av[:, 0:32] + av[:, 32:64], av[:, 64:96] + av[:, 96:128]],
            axis=1)
        t = dinv * (acc + pv[:, 0:64]) + br[...]
        t = jnp.maximum(t, 0.0)
        h = dinv * jnp.dot(t, wr[...], preferred_element_type=jnp.float32)
        o[...] = jnp.concatenate(
            [h, jnp.zeros((h.shape[0], 128 - F2), jnp.float32)], axis=1)

    return pl.pallas_call(
        body,
        grid=(NPAD // _BM,),
        in_specs=[_blk(128), _blk(128), _full(1, 64), _full(64, F2)],
        out_specs=_blk(128),
        out_shape=jax.ShapeDtypeStruct((NPAD, 128), jnp.float32),
    )(a, p1, b, W)


def _layer2(a, p2, p1, b, W):
    F = 32
    F2 = W.shape[1]

    def body(ar, p2r, p1r, br, wr, o):
        av = ar[...]
        dinv = p1r[...][:, 64:65]
        t = dinv * (av[:, 0:F] + av[:, F:2 * F] + p2r[...][:, 0:F]) + br[...]
        t = jnp.maximum(t, 0.0)
        h = dinv * jnp.dot(t, wr[...], preferred_element_type=jnp.float32)
        o[...] = jnp.concatenate(
            [h, jnp.zeros((h.shape[0], 128 - F2), jnp.float32)], axis=1)

    return pl.pallas_call(
        body,
        grid=(NPAD // _BM,),
        in_specs=[
            _blk(128), _blk(128), _blk(128),
            _full(1, F), _full(F, F2),
        ],
        out_specs=_blk(128),
        out_shape=jax.ShapeDtypeStruct((NPAD, 128), jnp.float32),
    )(a, p2, p1, b, W)


def _final(a, p3, p1, b):
    F = 16
    bm = 1000

    def body(ar, p3r, p1r, br, o):
        av = ar[...]
        dinv = p1r[...][:, 64:65]
        t = dinv * (av[:, 0:F] + av[:, F:2 * F] + p3r[...][:, 0:F]) + br[...]
        o[...] = jax.nn.sigmoid(t)

    return pl.pallas_call(
        body,
        grid=(N // bm,),
        in_specs=[
            _blk(128, bm), _blk(128, bm), _blk(128, bm),
            _full(1, F),
        ],
        out_specs=_blk(F, bm),
        out_shape=jax.ShapeDtypeStruct((N, F), jnp.float32),
    )(a, p3, p1, b)


def kernel(x, edge_index, W1, b1, W2, b2, W3, b3):
    pad = EPAD - E
    dst = jnp.concatenate(
        [edge_index[1].astype(jnp.int32),
         N + (jnp.arange(pad, dtype=jnp.int32) % (NPAD - N))])
    src = jnp.concatenate(
        [edge_index[0].astype(jnp.int32), jnp.zeros((pad,), jnp.int32)])
    src = src.reshape(NW, NCH, C)
    dst = dst.reshape(NW, NCH, C)
    xp = jnp.pad(x, ((0, NPAD - N), (0, 0)))

    degp = _deg_pass(dst).reshape(2 * NPAD, 1)
    p1 = _mm1s(xp, W1, degp)

    a = _prop64(p1, src, dst)
    p2 = _layer1(a, p1, b1.reshape(1, -1), W2)

    a = _prop32(p2, src, dst)
    p3 = _layer2(a, p2, p1, b2.reshape(1, -1), W3)

    a = _prop16(p3, src, dst)
    return _final(a, p3, p1, b3.reshape(1, -1))


# speedup vs baseline: 1.0215x; 1.0059x over previous
import functools

import jax
import jax.numpy as jnp
from jax import lax
from jax.experimental import pallas as pl
from jax.experimental.pallas import tpu as pltpu
from jax.experimental.pallas import tpu_sc as plsc

N = 10000
E = 320000
NC, NS = 2, 16
NW = NC * NS
C = 128
CH0 = 88
CH1 = 72
CHT = NS * (CH0 + CH1)
EPAD = (CHT + 16) * C
NPAD = 10240
RPT = NPAD // NS
NB = 4
_BM = 2048

_MESH = plsc.VectorSubcoreMesh(core_axis_name="c", subcore_axis_name="s")
_SC_PARAMS = pltpu.CompilerParams(use_tc_tiling_on_sc=False)


def _make_prop(FB, ncb):
    @functools.partial(
        pl.kernel,
        out_type=jax.ShapeDtypeStruct((NPAD, 128), jnp.float32),
        mesh=_MESH,
        compiler_params=_SC_PARAMS,
        scratch_types=[
            pltpu.VMEM((CH0, C), jnp.int32),
            pltpu.VMEM((CH0, C), jnp.int32),
            pltpu.VMEM((2, NB, C, FB), jnp.float32),
            pltpu.VMEM((C, FB), jnp.float32),
            pltpu.SemaphoreType.DMA,
            pltpu.SemaphoreType.DMA,
            pltpu.SemaphoreType.DMA,
            pltpu.SemaphoreType.DMA,
            pltpu.SemaphoreType.DMA,
            pltpu.VMEM_SHARED((NPAD, FB), jnp.float32),
            pltpu.VMEM_SHARED((NPAD, FB), jnp.float32),
        ],
    )
    def prop(h_hbm, src_hbm, dst_hbm, out_hbm, srcv, dstv, rows, zbuf,
             g0, g1, s0, s1, hsem, acc, hsp):
        cid = lax.axis_index("c")
        sid = lax.axis_index("s")
        gsem = (g0, g1)
        ssem = (s0, s1)
        base = jnp.where(cid == 0, sid * CH0, NS * CH0 + sid * CH1)
        ngrp = jnp.where(cid == 0, CH0 // NB, CH1 // NB)

        pltpu.sync_copy(src_hbm.at[pl.ds(base, CH0)], srcv)
        pltpu.sync_copy(dst_hbm.at[pl.ds(base, CH0)], dstv)

        def zrow(i, carry):
            for c4 in range(FB // 16):
                zbuf[i, pl.ds(c4 * 16, 16)] = jnp.zeros((16,), jnp.float32)
            return carry

        lax.fori_loop(0, C, zrow, 0)

        def fire_g(slot, grp):
            for b in range(NB):
                i = grp * NB + b
                pltpu.async_copy(hsp.at[srcv.at[i]], rows.at[slot, b],
                                 gsem[slot])

        def wait_g(slot, grp):
            for b in range(NB):
                i = grp * NB + b
                pltpu.make_async_copy(hsp.at[srcv.at[i]],
                                      rows.at[slot, b], gsem[slot]).wait()

        def run_s(slot, grp):
            ds = []
            for b in range(NB):
                i = grp * NB + b
                ds.append(pltpu.async_copy(rows.at[slot, b],
                                           acc.at[dstv.at[i]], ssem[slot],
                                           add=True))
            for d in ds:
                d.wait()

        for cb in range(ncb):
            stg = pltpu.async_copy(
                h_hbm.at[pl.ds(sid * RPT, RPT), pl.ds(cb * FB, FB)],
                hsp.at[pl.ds(sid * RPT, RPT)], hsem)
            for r in range(RPT // C):
                pltpu.sync_copy(zbuf, acc.at[pl.ds(sid * RPT + r * C, C)])
            stg.wait()
            plsc.subcore_barrier()

            fire_g(0, 0)
            fire_g(1, 1)

            def outer(j2, carry):
                j = j2 * 2
                wait_g(0, j)
                run_s(0, j)
                fire_g(0, lax.rem(j + 2, ngrp))
                wait_g(1, j + 1)
                run_s(1, j + 1)
                fire_g(1, lax.rem(j + 3, ngrp))
                return carry

            lax.fori_loop(0, ngrp // 2, outer, 0)
            wait_g(0, 0)
            wait_g(1, 1)
            plsc.subcore_barrier()
            pltpu.sync_copy(
                acc.at[pl.ds(sid * RPT, RPT)],
                out_hbm.at[pl.ds(sid * RPT, RPT),
                           pl.ds((cb * NC + cid) * FB, FB)],
            )
            if cb + 1 < ncb:
                plsc.subcore_barrier()

    return prop


_prop64 = _make_prop(32, 2)
_prop32 = _make_prop(32, 1)
_prop16 = _make_prop(16, 1)


@functools.partial(
    pl.kernel,
    out_type=jax.ShapeDtypeStruct((2 * NPAD,), jnp.float32),
    mesh=_MESH,
    compiler_params=_SC_PARAMS,
    scratch_types=[
        pltpu.VMEM((CH0, C), jnp.int32),
        pltpu.VMEM((C,), jnp.float32),
        pltpu.SemaphoreType.DMA,
        pltpu.VMEM_SHARED((NPAD,), jnp.float32),
    ],
)
def _deg_pass(dst_hbm, out_hbm, dstv, ones, sem, acc):
    cid = lax.axis_index("c")
    sid = lax.axis_index("s")
    base = jnp.where(cid == 0, sid * CH0, NS * CH0 + sid * CH1)
    nj = jnp.where(cid == 0, CH0 // 8, CH1 // 8)

    pltpu.sync_copy(dst_hbm.at[pl.ds(base, CH0)], dstv)
    for c4 in range(C // 16):
        ones[pl.ds(c4 * 16, 16)] = jnp.zeros((16,), jnp.float32)
    for r in range(RPT // C):
        pltpu.sync_copy(ones, acc.at[pl.ds(sid * RPT + r * C, C)])
    plsc.subcore_barrier()
    for c4 in range(C // 16):
        ones[pl.ds(c4 * 16, 16)] = jnp.ones((16,), jnp.float32)

    def body(j, carry):
        ds = []
        for b in range(8):
            i = j * 8 + b
            ds.append(pltpu.async_copy(ones, acc.at[dstv.at[i]], sem,
                                       add=True))
        for d in ds:
            d.wait()
        return carry

    lax.fori_loop(0, nj, body, 0)
    plsc.subcore_barrier()
    pltpu.sync_copy(
        acc.at[pl.ds(sid * RPT, RPT)],
        out_hbm.at[pl.ds(cid * NPAD + sid * RPT, RPT)],
    )


def _blk(w, bm=_BM):
    return pl.BlockSpec((bm, w), lambda i: (i, 0))


def _full(r, c):
    return pl.BlockSpec((r, c), lambda i: (0, 0))


def _mm1s(x, W, degp):
    M, K = x.shape
    F = W.shape[1]

    def body(xr, wr, d0r, d1r, o):
        dinv = lax.rsqrt(d0r[...] + d1r[...] + 1.0)
        h = dinv * jnp.dot(xr[...], wr[...],
                           preferred_element_type=jnp.float32)
        o[...] = jnp.concatenate(
            [h, dinv, jnp.zeros((h.shape[0], 128 - F - 1), jnp.float32)],
            axis=1)

    nsec = NPAD // _BM
    return pl.pallas_call(
        body,
        grid=(M // _BM,),
        in_specs=[
            _blk(K), _full(K, F),
            pl.BlockSpec((_BM, 1), lambda i: (i, 0)),
            pl.BlockSpec((_BM, 1), lambda i: (nsec + i, 0)),
        ],
        out_specs=_blk(128),
        out_shape=jax.ShapeDtypeStruct((M, 128), jnp.float32),
    )(x, W, degp, degp)


def _layer1(a, p1, b, W):
    F2 = W.shape[1]

    def body(ar, pr, br, wr, o):
        av = ar[...]
        pv = pr[...]
        dinv = pv[:, 64:65]
        acc = jnp.concatenate(
            [av[:, 0:32] + av[:, 32:64], av[:, 64:96] + av[:, 96:128]],
            axis=1)
        t = dinv * (acc + pv[:, 0:64]) + br[...]
        t = jnp.maximum(t, 0.0)
        h = dinv * jnp.dot(t, wr[...], preferred_element_type=jnp.float32)
        o[...] = jnp.concatenate(
            [h, jnp.zeros((h.shape[0], 128 - F2), jnp.float32)], axis=1)

    return pl.pallas_call(
        body,
        grid=(NPAD // _BM,),
        in_specs=[_blk(128), _blk(128), _full(1, 64), _full(64, F2)],
        out_specs=_blk(128),
        out_shape=jax.ShapeDtypeStruct((NPAD, 128), jnp.float32),
    )(a, p1, b, W)


def _layer2(a, p2, p1, b, W):
    F = 32
    F2 = W.shape[1]

    def body(ar, p2r, p1r, br, wr, o):
        av = ar[...]
        dinv = p1r[...][:, 64:65]
        t = dinv * (av[:, 0:F] + av[:, F:2 * F] + p2r[...][:, 0:F]) + br[...]
        t = jnp.maximum(t, 0.0)
        h = dinv * jnp.dot(t, wr[...], preferred_element_type=jnp.float32)
        o[...] = jnp.concatenate(
            [h, jnp.zeros((h.shape[0], 128 - F2), jnp.float32)], axis=1)

    return pl.pallas_call(
        body,
        grid=(NPAD // _BM,),
        in_specs=[
            _blk(128), _blk(128), _blk(128),
            _full(1, F), _full(F, F2),
        ],
        out_specs=_blk(128),
        out_shape=jax.ShapeDtypeStruct((NPAD, 128), jnp.float32),
    )(a, p2, p1, b, W)


def _final(a, p3, p1, b):
    F = 16
    bm = 1000

    def body(ar, p3r, p1r, br, o):
        av = ar[...]
        dinv = p1r[...][:, 64:65]
        t = dinv * (av[:, 0:F] + av[:, F:2 * F] + p3r[...][:, 0:F]) + br[...]
        o[...] = jax.nn.sigmoid(t)

    return pl.pallas_call(
        body,
        grid=(N // bm,),
        in_specs=[
            _blk(128, bm), _blk(128, bm), _blk(128, bm),
            _full(1, F),
        ],
        out_specs=_blk(F, bm),
        out_shape=jax.ShapeDtypeStruct((N, F), jnp.float32),
    )(a, p3, p1, b)


def kernel(x, edge_index, W1, b1, W2, b2, W3, b3):
    pad = EPAD - E
    dst = jnp.concatenate(
        [edge_index[1].astype(jnp.int32),
         N + (jnp.arange(pad, dtype=jnp.int32) % (NPAD - N))])
    src = jnp.concatenate(
        [edge_index[0].astype(jnp.int32), jnp.zeros((pad,), jnp.int32)])
    src = src.reshape(EPAD // C, C)
    dst = dst.reshape(EPAD // C, C)
    xp = jnp.pad(x, ((0, NPAD - N), (0, 0)))

    degp = _deg_pass(dst).reshape(2 * NPAD, 1)
    p1 = _mm1s(xp, W1, degp)

    a = _prop64(p1, src, dst)
    p2 = _layer1(a, p1, b1.reshape(1, -1), W2)

    a = _prop32(p2, src, dst)
    p3 = _layer2(a, p2, p1, b2.reshape(1, -1), W3)

    a = _prop16(p3, src, dst)
    return _final(a, p3, p1, b3.reshape(1, -1))
